# baseline (speedup 1.0000x reference)
import functools

import jax
import jax.numpy as jnp
from jax import lax
from jax.experimental import pallas as pl
from jax.experimental.pallas import tpu as pltpu
from jax.experimental.pallas import tpu_sc as plsc

_N = 10000
_E = 320000
_B = 125
_NC, _NS = 2, 16
_TILES = _NC * _NS
_EPT = _E // _TILES
_CH = _EPT // _B
_RPS = _N // _NS
_ZCH = _RPS // _B
_BM = 1000
_GRID = _N // _BM


_DST = 2000


@functools.lru_cache(maxsize=None)
def _sc_agg(W):
  mesh = plsc.VectorSubcoreMesh(
      core_axis_name="c", subcore_axis_name="s", num_cores=_NC, num_subcores=_NS)

  out_type = (jax.ShapeDtypeStruct((4, _NC, _N, W), jnp.float32),
              jax.ShapeDtypeStruct((_NC, _N), jnp.float32))
  scratch = [
      pltpu.VMEM((_CH, _B), jnp.int32),
      pltpu.VMEM((_CH, _B), jnp.int32),
      pltpu.VMEM((4, _B, W), jnp.float32),
      pltpu.VMEM((_B, W), jnp.float32),
      pltpu.VMEM_SHARED((_N, W), jnp.float32),
      pltpu.SemaphoreType.DMA,
      pltpu.SemaphoreType.DMA,
      pltpu.SemaphoreType.DMA,
      pltpu.SemaphoreType.DMA,
      pltpu.SemaphoreType.DMA,
      pltpu.SemaphoreType.DMA,
      pltpu.SemaphoreType.DMA,
      pltpu.SemaphoreType.DMA,
      pltpu.VMEM((16,), jnp.int32),
      pltpu.VMEM((16,), jnp.int32),
      pltpu.VMEM((128,), jnp.float32),
      pltpu.VMEM((_DST,), jnp.float32),
      pltpu.VMEM_SHARED((_N,), jnp.float32),
  ]

  @functools.partial(
      pl.kernel,
      out_type=out_type,
      mesh=mesh,
      scratch_types=scratch,
      compiler_params=pltpu.CompilerParams(
          use_tc_tiling_on_sc=False, needs_layout_passes=False),
  )
  def k(src_hbm, dst_hbm, table_hbm, cnt_hbm, dflag_hbm, *rest):
    (out_hbm, deg_hbm, src_v, dst_v, rows_v, stage_v, acc_sh,
     g0, g1, g2, g3, s0, s1, s2, s3,
     cnt_v, dflag_v, ones_v, dstage_v, dacc_sh) = rest
    gsem = (g0, g1, g2, g3)
    ssem = (s0, s1, s2, s3)
    c = lax.axis_index("c")
    s = lax.axis_index("s")
    wid = s * _NC + c
    pltpu.sync_copy(src_hbm.at[pl.ds(wid * _CH, _CH)], src_v)
    pltpu.sync_copy(dst_hbm.at[pl.ds(wid * _CH, _CH)], dst_v)
    pltpu.sync_copy(cnt_hbm, cnt_v)
    pltpu.sync_copy(dflag_hbm, dflag_v)
    cnt = lax.shift_right_logical(jnp.sum(cnt_v[...]), 4)
    have_deg = jnp.sum(dflag_v[...]) > 0

    zero16 = jnp.zeros((16,), jnp.float32)

    def zrow(i, carry):
      for j in range(W // 16):
        stage_v[i, pl.ds(j * 16, 16)] = zero16
      return carry

    lax.fori_loop(0, _B, zrow, 0)

    @pl.when(have_deg)
    def _():
      one16 = jnp.ones((16,), jnp.float32)

      def fill1(i, carry):
        ones_v[pl.ds(i * 16, 16)] = one16
        return carry

      lax.fori_loop(0, 128 // 16, fill1, 0)

      def fill0(i, carry):
        dstage_v[pl.ds(i * 16, 16)] = zero16
        return carry

      lax.fori_loop(0, _DST // 16, fill0, 0)

      @pl.when(s < _N // _DST)
      def _():
        pltpu.sync_copy(dstage_v, dacc_sh.at[pl.ds(s * _DST, _DST)])

    def phase(kk, carry):
      tbl = table_hbm.at[kk]

      def zcp(i, carry2):
        pltpu.sync_copy(stage_v, acc_sh.at[pl.ds(s * _RPS + i * _B, _B)])
        return carry2

      lax.fori_loop(0, _ZCH, zcp, 0)
      plsc.subcore_barrier()

      def issue_gather(g, j):
        pltpu.async_copy(tbl.at[src_v.at[g]], rows_v.at[j], gsem[j])

      def wait_gather(g, j):
        pltpu.make_async_copy(
            tbl.at[src_v.at[g]], rows_v.at[j], gsem[j]).wait()

      def start_scatter(g, j):
        pltpu.async_copy(rows_v.at[j], acc_sh.at[dst_v.at[g]], ssem[j],
                         add=True)

      def wait_scatter(g, j):
        pltpu.make_async_copy(rows_v.at[j], acc_sh.at[dst_v.at[g]],
                              ssem[j]).wait()

      for j in range(4):
        issue_gather(j, j)

      def body(i, carry2):
        g = 4 * i
        for j in range(4):
          wait_gather(g + j, j)
          start_scatter(g + j, j)
        for j in range(4):
          wait_scatter(g + j, j)
          issue_gather(g + 4 + j, j)
        return carry2

      lax.fori_loop(0, _CH // 4 - 1, body, 0)
      gl = _CH - 4
      for j in range(4):
        wait_gather(gl + j, j)
        start_scatter(gl + j, j)
      for j in range(4):
        wait_scatter(gl + j, j)

      @pl.when(jnp.logical_and(have_deg, kk == 0))
      def _():
        def dbody(g, carry2):
          pltpu.sync_copy(ones_v.at[pl.ds(0, _B)], dacc_sh.at[dst_v.at[g]],
                          add=True)
          return carry2

        lax.fori_loop(0, _CH, dbody, 0)

      plsc.subcore_barrier()

      pltpu.sync_copy(acc_sh.at[pl.ds(s * _RPS, _RPS)],
                      out_hbm.at[kk, c, pl.ds(s * _RPS, _RPS)])
      return carry

    lax.fori_loop(0, cnt, phase, 0)

    @pl.when(jnp.logical_and(have_deg, s < _N // _DST))
    def _():
      pltpu.sync_copy(dacc_sh.at[pl.ds(s * _DST, _DST)],
                      deg_hbm.at[c, pl.ds(s * _DST, _DST)])

  return k


def _rows(d):
  return pl.BlockSpec((_BM, d), lambda i: (i, 0))


def _part(d):
  return pl.BlockSpec((_NC, _BM, d), lambda i: (0, i, 0))


def _full(r, c):
  return pl.BlockSpec((r, c), lambda i: (0, 0))


def _out(d):
  return jax.ShapeDtypeStruct((_N, d), jnp.float32)


def _tbl_spec():
  return pl.BlockSpec((4, _BM, 64), lambda i: (0, i, 0))


def _tbl_out():
  return jax.ShapeDtypeStruct((4, _N, 64), jnp.float32)


def _mm_body(x_ref, w_ref, o_ref):
  o_ref[0, :, :] = jnp.dot(x_ref[...], w_ref[...],
                           preferred_element_type=jnp.float32)


def _tc_mm(x, w):
  fin, fout = w.shape
  return pl.pallas_call(
      _mm_body,
      grid=(_GRID,),
      in_specs=[_rows(fin), _full(fin, fout)],
      out_specs=_tbl_spec(),
      out_shape=_tbl_out(),
  )(x, w)


def _mm_bias_body(*refs):
  npairs = (len(refs) - 2) // 2
  hs, ws, b, o = refs[:npairs], refs[npairs:2 * npairs], refs[-2], refs[-1]
  acc = b[...]
  for h, w in zip(hs, ws):
    acc = acc + jnp.dot(h[...], w[...], preferred_element_type=jnp.float32)
  o[...] = acc


def _tc_mm_bias(hs, ws, b, fout):
  return pl.pallas_call(
      _mm_bias_body,
      grid=(_GRID,),
      in_specs=[_rows(h.shape[1]) for h in hs]
               + [_full(w.shape[0], fout) for w in ws] + [_full(1, fout)],
      out_specs=_rows(fout),
      out_shape=_out(fout),
  )(*hs, *ws, b)


def _c0_body(degp, p, g, wl1, h1_o, t1_o, inv_o):
  deg = degp[0] + degp[1]
  inv = 1.0 / jnp.maximum(deg, 1.0)
  h1 = jnp.maximum((p[0] + p[1]) * inv + g[...], 0.0)
  h1_o[...] = h1
  t1_o[0, :, :] = jnp.dot(h1, wl1[...], preferred_element_type=jnp.float32)
  inv_o[...] = inv


def _c1_body(p, inv, g, o):
  o[0, :, :] = jnp.maximum((p[0] + p[1]) * inv[...] + g[...], 0.0)


def _c2_body(p, inv, g, wl, o):
  agg = (p[0] + p[1]) * inv[...]
  h3 = jnp.maximum(
      jnp.dot(agg, wl[...], preferred_element_type=jnp.float32) + g[...], 0.0)
  o[0, :, :] = h3[:, :64]
  o[1, :, :] = h3[:, 64:]


def _c3_body(pa, pb, inv, g, wla, wlb, o):
  iv = inv[...]
  h4 = jnp.maximum(
      jnp.dot((pa[0] + pa[1]) * iv, wla[...], preferred_element_type=jnp.float32)
      + jnp.dot((pb[0] + pb[1]) * iv, wlb[...], preferred_element_type=jnp.float32)
      + g[...], 0.0)
  o[0, :, :] = h4[:, :64]
  o[1, :, :] = h4[:, 64:128]
  o[2, :, :] = h4[:, 128:192]
  o[3, :, :] = h4[:, 192:]


def _c4_body(pa, pb, pc, pd, inv, g, wla, wlb, wlc, wld, wout, bout, o):
  iv = inv[...]
  h5 = (jnp.dot((pa[0] + pa[1]) * iv, wla[...], preferred_element_type=jnp.float32)
        + jnp.dot((pb[0] + pb[1]) * iv, wlb[...], preferred_element_type=jnp.float32)
        + jnp.dot((pc[0] + pc[1]) * iv, wlc[...], preferred_element_type=jnp.float32)
        + jnp.dot((pd[0] + pd[1]) * iv, wld[...], preferred_element_type=jnp.float32)
        + g[...])
  h5 = jnp.maximum(h5, 0.0)
  logits = jnp.dot(h5, wout[...], preferred_element_type=jnp.float32) + bout[...]
  m = jnp.max(logits, axis=-1, keepdims=True)
  e = jnp.exp(logits - m)
  o[...] = e / jnp.sum(e, axis=-1, keepdims=True)


def kernel(x, edge_index, Wl0, Wr0, b0, Wl1, Wr1, b1, Wl2, Wr2, b2,
           Wl3, Wr3, b3, Wl4, Wr4, b4, Wout, bout):
  src2 = edge_index[0].reshape(_E // _B, _B)
  dst2 = edge_index[1].reshape(_E // _B, _B)
  b0r, b1r, b2r, b3r, b4r = (v.reshape(1, -1) for v in (b0, b1, b2, b3, b4))
  boutr = bout.reshape(1, -1)

  f1 = jnp.ones((16,), jnp.int32)
  f0 = jnp.zeros((16,), jnp.int32)
  c1 = jnp.full((16,), 1, jnp.int32)
  c2 = jnp.full((16,), 2, jnp.int32)
  c4 = jnp.full((16,), 4, jnp.int32)
  agg = _sc_agg(64)

  t0 = _tc_mm(x, Wl0)
  P0, degp = agg(src2, dst2, t0, c1, f1)
  g0 = _tc_mm_bias([x], [Wr0], b0r, 64)
  degp = degp.reshape(_NC, _N, 1)
  h1, t1, inv = pl.pallas_call(
      _c0_body,
      grid=(_GRID,),
      in_specs=[_part(1), _part(64), _rows(64), _full(64, 64)],
      out_specs=[_rows(64), _tbl_spec(), _rows(1)],
      out_shape=[_out(64), _tbl_out(), _out(1)],
  )(degp, P0[0], g0, Wl1)

  P1, _ = agg(src2, dst2, t1, c1, f0)
  g1 = _tc_mm_bias([h1], [Wr1], b1r, 64)
  t2 = pl.pallas_call(
      _c1_body,
      grid=(_GRID,),
      in_specs=[_part(64), _rows(1), _rows(64)],
      out_specs=_tbl_spec(),
      out_shape=_tbl_out(),
  )(P1[0], inv, g1)

  h2 = t2[0]
  P2, _ = agg(src2, dst2, t2, c1, f0)
  g2 = _tc_mm_bias([h2], [Wr2], b2r, 128)
  t3 = pl.pallas_call(
      _c2_body,
      grid=(_GRID,),
      in_specs=[_part(64), _rows(1), _rows(128), _full(64, 128)],
      out_specs=_tbl_spec(),
      out_shape=_tbl_out(),
  )(P2[0], inv, g2, Wl2)

  P3, _ = agg(src2, dst2, t3, c2, f0)
  g3 = _tc_mm_bias([t3[0], t3[1]], [Wr3[:64], Wr3[64:]], b3r, 256)
  t4 = pl.pallas_call(
      _c3_body,
      grid=(_GRID,),
      in_specs=[_part(64), _part(64), _rows(1), _rows(256),
                _full(64, 256), _full(64, 256)],
      out_specs=_tbl_spec(),
      out_shape=_tbl_out(),
  )(P3[0], P3[1], inv, g3, Wl3[:64], Wl3[64:])

  P4, _ = agg(src2, dst2, t4, c4, f0)
  g4 = _tc_mm_bias([t4[0], t4[1], t4[2], t4[3]],
                   [Wr4[:64], Wr4[64:128], Wr4[128:192], Wr4[192:]],
                   b4r, 512)
  out = pl.pallas_call(
      _c4_body,
      grid=(_GRID,),
      in_specs=[_part(64)] * 4 + [_rows(1), _rows(512)]
               + [_full(64, 512)] * 4 + [_full(512, 4), _full(1, 4)],
      out_specs=_rows(4),
      out_shape=jax.ShapeDtypeStruct((_N, 4), jnp.float32),
  )(P4[0], P4[1], P4[2], P4[3], inv, g4,
    Wl4[:64], Wl4[64:128], Wl4[128:192], Wl4[192:],
    Wout, boutr)
  return out


# speedup vs baseline: 1.2788x; 1.2788x over previous
import functools

import jax
import jax.numpy as jnp
from jax import lax
from jax.experimental import pallas as pl
from jax.experimental.pallas import tpu as pltpu
from jax.experimental.pallas import tpu_sc as plsc

_N = 10000
_E = 320000
_B = 125
_NC, _NS = 2, 16
_TILES = _NC * _NS
_EPT = _E // _TILES
_CH = _EPT // _B
_RPS = _N // _NS
_ZCH = _RPS // _B
_BM = 1000
_GRID = _N // _BM


_DST = 2000


@functools.lru_cache(maxsize=None)
def _sc_agg(W):
  mesh = plsc.VectorSubcoreMesh(
      core_axis_name="c", subcore_axis_name="s", num_cores=_NC, num_subcores=_NS)

  out_type = (jax.ShapeDtypeStruct((_NC, _N, W), jnp.float32),
              jax.ShapeDtypeStruct((_NC, _N), jnp.float32))
  scratch = [
      pltpu.VMEM((_CH, _B), jnp.int32),
      pltpu.VMEM((_CH, _B), jnp.int32),
      pltpu.VMEM((4, _B, W), jnp.float32),
      pltpu.VMEM((_B, W), jnp.float32),
      pltpu.VMEM_SHARED((_N, W), jnp.float32),
      pltpu.SemaphoreType.DMA,
      pltpu.SemaphoreType.DMA,
      pltpu.SemaphoreType.DMA,
      pltpu.SemaphoreType.DMA,
      pltpu.SemaphoreType.DMA,
      pltpu.SemaphoreType.DMA,
      pltpu.SemaphoreType.DMA,
      pltpu.SemaphoreType.DMA,
      pltpu.VMEM((16,), jnp.int32),
      pltpu.VMEM((128,), jnp.float32),
      pltpu.VMEM((_DST,), jnp.float32),
      pltpu.VMEM_SHARED((_N,), jnp.float32),
  ]

  @functools.partial(
      pl.kernel,
      out_type=out_type,
      mesh=mesh,
      scratch_types=scratch,
      compiler_params=pltpu.CompilerParams(
          use_tc_tiling_on_sc=False, needs_layout_passes=False),
  )
  def k(src_hbm, dst_hbm, table_hbm, dflag_hbm, *rest):
    (out_hbm, deg_hbm, src_v, dst_v, rows_v, stage_v, acc_sh,
     g0, g1, g2, g3, s0, s1, s2, s3,
     dflag_v, ones_v, dstage_v, dacc_sh) = rest
    gsem = (g0, g1, g2, g3)
    ssem = (s0, s1, s2, s3)
    c = lax.axis_index("c")
    s = lax.axis_index("s")
    wid = s * _NC + c
    pltpu.sync_copy(src_hbm.at[pl.ds(wid * _CH, _CH)], src_v)
    pltpu.sync_copy(dst_hbm.at[pl.ds(wid * _CH, _CH)], dst_v)
    pltpu.sync_copy(dflag_hbm, dflag_v)
    have_deg = jnp.sum(dflag_v[...]) > 0

    zero16 = jnp.zeros((16,), jnp.float32)

    def zrow(i, carry):
      for j in range(W // 16):
        stage_v[i, pl.ds(j * 16, 16)] = zero16
      return carry

    lax.fori_loop(0, _B, zrow, 0)

    @pl.when(have_deg)
    def _():
      one16 = jnp.ones((16,), jnp.float32)

      def fill1(i, carry):
        ones_v[pl.ds(i * 16, 16)] = one16
        return carry

      lax.fori_loop(0, 128 // 16, fill1, 0)

      def fill0(i, carry):
        dstage_v[pl.ds(i * 16, 16)] = zero16
        return carry

      lax.fori_loop(0, _DST // 16, fill0, 0)

      @pl.when(s < _N // _DST)
      def _():
        pltpu.sync_copy(dstage_v, dacc_sh.at[pl.ds(s * _DST, _DST)])

    def zcp(i, carry2):
      pltpu.sync_copy(stage_v, acc_sh.at[pl.ds(s * _RPS + i * _B, _B)])
      return carry2

    lax.fori_loop(0, _ZCH, zcp, 0)
    plsc.subcore_barrier()

    def issue_gather(g, j):
      pltpu.async_copy(table_hbm.at[src_v.at[g]], rows_v.at[j], gsem[j])

    def wait_gather(g, j):
      pltpu.make_async_copy(
          table_hbm.at[src_v.at[g]], rows_v.at[j], gsem[j]).wait()

    def start_scatter(g, j):
      pltpu.async_copy(rows_v.at[j], acc_sh.at[dst_v.at[g]], ssem[j],
                       add=True)

    def wait_scatter(g, j):
      pltpu.make_async_copy(rows_v.at[j], acc_sh.at[dst_v.at[g]],
                            ssem[j]).wait()

    for j in range(4):
      issue_gather(j, j)

    def body(i, carry2):
      g = 4 * i
      for j in range(4):
        wait_gather(g + j, j)
        start_scatter(g + j, j)
      for j in range(4):
        wait_scatter(g + j, j)
        issue_gather(g + 4 + j, j)
      return carry2

    lax.fori_loop(0, _CH // 4 - 1, body, 0)
    gl = _CH - 4
    for j in range(4):
      wait_gather(gl + j, j)
      start_scatter(gl + j, j)
    for j in range(4):
      wait_scatter(gl + j, j)

    @pl.when(have_deg)
    def _():
      def dbody(g, carry2):
        pltpu.sync_copy(ones_v.at[pl.ds(0, _B)], dacc_sh.at[dst_v.at[g]],
                        add=True)
        return carry2

      lax.fori_loop(0, _CH, dbody, 0)

    plsc.subcore_barrier()

    pltpu.sync_copy(acc_sh.at[pl.ds(s * _RPS, _RPS)],
                    out_hbm.at[c, pl.ds(s * _RPS, _RPS)])

    @pl.when(jnp.logical_and(have_deg, s < _N // _DST))
    def _():
      pltpu.sync_copy(dacc_sh.at[pl.ds(s * _DST, _DST)],
                      deg_hbm.at[c, pl.ds(s * _DST, _DST)])

  return k


def _rows(d):
  return pl.BlockSpec((_BM, d), lambda i: (i, 0))


def _part(d):
  return pl.BlockSpec((_NC, _BM, d), lambda i: (0, i, 0))


def _full(r, c):
  return pl.BlockSpec((r, c), lambda i: (0, 0))


def _out(d):
  return jax.ShapeDtypeStruct((_N, d), jnp.float32)


def _mm_body(x_ref, w_ref, o_ref):
  o_ref[...] = jnp.dot(x_ref[...], w_ref[...],
                       preferred_element_type=jnp.float32)


def _tc_mm(x, w):
  fin, fout = w.shape
  return pl.pallas_call(
      _mm_body,
      grid=(_GRID,),
      in_specs=[_rows(fin), _full(fin, fout)],
      out_specs=_rows(fout),
      out_shape=_out(fout),
  )(x, w)


def _mm_bias_body(*refs):
  npairs = (len(refs) - 2) // 2
  hs, ws, b, o = refs[:npairs], refs[npairs:2 * npairs], refs[-2], refs[-1]
  acc = b[...]
  for h, w in zip(hs, ws):
    acc = acc + jnp.dot(h[...], w[...], preferred_element_type=jnp.float32)
  o[...] = acc


def _tc_mm_bias(hs, ws, b, fout):
  return pl.pallas_call(
      _mm_bias_body,
      grid=(_GRID,),
      in_specs=[_rows(h.shape[1]) for h in hs]
               + [_full(w.shape[0], fout) for w in ws] + [_full(1, fout)],
      out_specs=_rows(fout),
      out_shape=_out(fout),
  )(*hs, *ws, b)


def _c0_body(degp, p, g, wl1, h1_o, y1_o, inv_o):
  deg = degp[0] + degp[1]
  inv = 1.0 / jnp.maximum(deg, 1.0)
  h1 = jnp.maximum((p[0] + p[1]) * inv + g[...], 0.0)
  h1_o[...] = h1
  y1_o[...] = jnp.dot(h1, wl1[...], preferred_element_type=jnp.float32)
  inv_o[...] = inv


def _c1_body(p, inv, g, o):
  o[...] = jnp.maximum((p[0] + p[1]) * inv[...] + g[...], 0.0)


def _c2_body(p, inv, g, wl, oa, ob):
  agg = (p[0] + p[1]) * inv[...]
  h3 = jnp.maximum(
      jnp.dot(agg, wl[...], preferred_element_type=jnp.float32) + g[...], 0.0)
  oa[...] = h3[:, :64]
  ob[...] = h3[:, 64:]


def _c3_body(pa, pb, inv, g, wla, wlb, oa, ob, oc, od):
  iv = inv[...]
  h4 = jnp.maximum(
      jnp.dot((pa[0] + pa[1]) * iv, wla[...], preferred_element_type=jnp.float32)
      + jnp.dot((pb[0] + pb[1]) * iv, wlb[...], preferred_element_type=jnp.float32)
      + g[...], 0.0)
  oa[...] = h4[:, :64]
  ob[...] = h4[:, 64:128]
  oc[...] = h4[:, 128:192]
  od[...] = h4[:, 192:]


def _c4_body(pa, pb, pc, pd, inv, g, wla, wlb, wlc, wld, wout, bout, o):
  iv = inv[...]
  h5 = (jnp.dot((pa[0] + pa[1]) * iv, wla[...], preferred_element_type=jnp.float32)
        + jnp.dot((pb[0] + pb[1]) * iv, wlb[...], preferred_element_type=jnp.float32)
        + jnp.dot((pc[0] + pc[1]) * iv, wlc[...], preferred_element_type=jnp.float32)
        + jnp.dot((pd[0] + pd[1]) * iv, wld[...], preferred_element_type=jnp.float32)
        + g[...])
  h5 = jnp.maximum(h5, 0.0)
  logits = jnp.dot(h5, wout[...], preferred_element_type=jnp.float32) + bout[...]
  m = jnp.max(logits, axis=-1, keepdims=True)
  e = jnp.exp(logits - m)
  o[...] = e / jnp.sum(e, axis=-1, keepdims=True)


def kernel(x, edge_index, Wl0, Wr0, b0, Wl1, Wr1, b1, Wl2, Wr2, b2,
           Wl3, Wr3, b3, Wl4, Wr4, b4, Wout, bout):
  src2 = edge_index[0].reshape(_E // _B, _B)
  dst2 = edge_index[1].reshape(_E // _B, _B)
  b0r, b1r, b2r, b3r, b4r = (v.reshape(1, -1) for v in (b0, b1, b2, b3, b4))
  boutr = bout.reshape(1, -1)

  f1 = jnp.ones((16,), jnp.int32)
  f0 = jnp.zeros((16,), jnp.int32)
  agg = _sc_agg(64)

  y0 = _tc_mm(x, Wl0)
  p0, degp = agg(src2, dst2, y0, f1)
  g0 = _tc_mm_bias([x], [Wr0], b0r, 64)
  degp = degp.reshape(_NC, _N, 1)
  h1, y1, inv = pl.pallas_call(
      _c0_body,
      grid=(_GRID,),
      in_specs=[_part(1), _part(64), _rows(64), _full(64, 64)],
      out_specs=[_rows(64), _rows(64), _rows(1)],
      out_shape=[_out(64), _out(64), _out(1)],
  )(degp, p0, g0, Wl1)

  p1, _ = agg(src2, dst2, y1, f0)
  g1 = _tc_mm_bias([h1], [Wr1], b1r, 64)
  h2 = pl.pallas_call(
      _c1_body,
      grid=(_GRID,),
      in_specs=[_part(64), _rows(1), _rows(64)],
      out_specs=_rows(64),
      out_shape=_out(64),
  )(p1, inv, g1)

  p2, _ = agg(src2, dst2, h2, f0)
  g2 = _tc_mm_bias([h2], [Wr2], b2r, 128)
  h3a, h3b = pl.pallas_call(
      _c2_body,
      grid=(_GRID,),
      in_specs=[_part(64), _rows(1), _rows(128), _full(64, 128)],
      out_specs=[_rows(64), _rows(64)],
      out_shape=[_out(64), _out(64)],
  )(p2, inv, g2, Wl2)

  p3a, _ = agg(src2, dst2, h3a, f0)
  p3b, _ = agg(src2, dst2, h3b, f0)
  g3 = _tc_mm_bias([h3a, h3b], [Wr3[:64], Wr3[64:]], b3r, 256)
  h4 = pl.pallas_call(
      _c3_body,
      grid=(_GRID,),
      in_specs=[_part(64), _part(64), _rows(1), _rows(256),
                _full(64, 256), _full(64, 256)],
      out_specs=[_rows(64)] * 4,
      out_shape=[_out(64)] * 4,
  )(p3a, p3b, inv, g3, Wl3[:64], Wl3[64:])

  p4 = [agg(src2, dst2, hq, f0)[0] for hq in h4]
  g4 = _tc_mm_bias(list(h4), [Wr4[:64], Wr4[64:128], Wr4[128:192],
                              Wr4[192:]], b4r, 512)
  out = pl.pallas_call(
      _c4_body,
      grid=(_GRID,),
      in_specs=[_part(64)] * 4 + [_rows(1), _rows(512)]
               + [_full(64, 512)] * 4 + [_full(512, 4), _full(1, 4)],
      out_specs=_rows(4),
      out_shape=jax.ShapeDtypeStruct((_N, 4), jnp.float32),
  )(p4[0], p4[1], p4[2], p4[3], inv, g4,
    Wl4[:64], Wl4[64:128], Wl4[128:192], Wl4[192:],
    Wout, boutr)
  return out


# speedup vs baseline: 1.3039x; 1.0196x over previous
import functools

import jax
import jax.numpy as jnp
from jax import lax
from jax.experimental import pallas as pl
from jax.experimental.pallas import tpu as pltpu
from jax.experimental.pallas import tpu_sc as plsc

_N = 10000
_E = 320000
_B = 125
_NC, _NS = 2, 16
_TILES = _NC * _NS
_EPT = _E // _TILES
_CH = _EPT // _B
_RPS = _N // _NS
_ZCH = _RPS // _B
_BM = 1000
_GRID = _N // _BM


_DST = 2000


@functools.lru_cache(maxsize=None)
def _sc_agg(W):
  mesh = plsc.VectorSubcoreMesh(
      core_axis_name="c", subcore_axis_name="s", num_cores=_NC, num_subcores=_NS)

  out_type = (jax.ShapeDtypeStruct((_NC, _N, W), jnp.float32),
              jax.ShapeDtypeStruct((_NC, _N), jnp.float32))
  scratch = [
      pltpu.VMEM((_CH, _B), jnp.int32),
      pltpu.VMEM((_CH, _B), jnp.int32),
      pltpu.VMEM((4, _B, W), jnp.float32),
      pltpu.VMEM((_B, W), jnp.float32),
      pltpu.VMEM_SHARED((_N, W), jnp.float32),
      pltpu.SemaphoreType.DMA,
      pltpu.SemaphoreType.DMA,
      pltpu.SemaphoreType.DMA,
      pltpu.SemaphoreType.DMA,
      pltpu.SemaphoreType.DMA,
      pltpu.SemaphoreType.DMA,
      pltpu.SemaphoreType.DMA,
      pltpu.SemaphoreType.DMA,
      pltpu.VMEM((16,), jnp.int32),
      pltpu.VMEM((128,), jnp.float32),
      pltpu.VMEM((_DST,), jnp.float32),
      pltpu.VMEM_SHARED((_N,), jnp.float32),
  ]

  @functools.partial(
      pl.kernel,
      out_type=out_type,
      mesh=mesh,
      scratch_types=scratch,
      compiler_params=pltpu.CompilerParams(
          use_tc_tiling_on_sc=False, needs_layout_passes=False),
  )
  def k(src_hbm, dst_hbm, table_hbm, dflag_hbm, *rest):
    (out_hbm, deg_hbm, src_v, dst_v, rows_v, stage_v, acc_sh,
     g0, g1, g2, g3, s0, s1, s2, s3,
     dflag_v, ones_v, dstage_v, dacc_sh) = rest
    gsem = (g0, g1, g2, g3)
    ssem = (s0, s1, s2, s3)
    c = lax.axis_index("c")
    s = lax.axis_index("s")
    wid = s * _NC + c
    pltpu.sync_copy(src_hbm.at[pl.ds(wid * _CH, _CH)], src_v)
    pltpu.sync_copy(dst_hbm.at[pl.ds(wid * _CH, _CH)], dst_v)
    pltpu.sync_copy(dflag_hbm, dflag_v)
    have_deg = jnp.sum(dflag_v[...]) > 0

    zero16 = jnp.zeros((16,), jnp.float32)

    def zrow(i, carry):
      for j in range(W // 16):
        stage_v[i, pl.ds(j * 16, 16)] = zero16
      return carry

    lax.fori_loop(0, _B, zrow, 0)

    @pl.when(have_deg)
    def _():
      one16 = jnp.ones((16,), jnp.float32)

      def fill1(i, carry):
        ones_v[pl.ds(i * 16, 16)] = one16
        return carry

      lax.fori_loop(0, 128 // 16, fill1, 0)

      def fill0(i, carry):
        dstage_v[pl.ds(i * 16, 16)] = zero16
        return carry

      lax.fori_loop(0, _DST // 16, fill0, 0)

      @pl.when(s < _N // _DST)
      def _():
        pltpu.sync_copy(dstage_v, dacc_sh.at[pl.ds(s * _DST, _DST)])

    def issue_gather(g, j):
      pltpu.async_copy(table_hbm.at[src_v.at[g]], rows_v.at[j], gsem[j])

    def wait_gather(g, j):
      pltpu.make_async_copy(
          table_hbm.at[src_v.at[g]], rows_v.at[j], gsem[j]).wait()

    def start_scatter(g, j):
      pltpu.async_copy(rows_v.at[j], acc_sh.at[dst_v.at[g]], ssem[j],
                       add=True)

    def wait_scatter(g, j):
      pltpu.make_async_copy(rows_v.at[j], acc_sh.at[dst_v.at[g]],
                            ssem[j]).wait()

    for j in range(4):
      issue_gather(j, j)

    def zcp(i, carry2):
      pltpu.sync_copy(stage_v, acc_sh.at[pl.ds(s * _RPS + i * _B, _B)])
      return carry2

    lax.fori_loop(0, _ZCH, zcp, 0)
    plsc.subcore_barrier()

    def body(i, carry2):
      g = 4 * i
      for j in range(4):
        wait_gather(g + j, j)
        start_scatter(g + j, j)
      for j in range(4):
        wait_scatter(g + j, j)
        issue_gather(g + 4 + j, j)
      return carry2

    lax.fori_loop(0, _CH // 4 - 1, body, 0)
    gl = _CH - 4
    for j in range(4):
      wait_gather(gl + j, j)
      start_scatter(gl + j, j)
    for j in range(4):
      wait_scatter(gl + j, j)

    @pl.when(have_deg)
    def _():
      def dbody(g, carry2):
        pltpu.sync_copy(ones_v.at[pl.ds(0, _B)], dacc_sh.at[dst_v.at[g]],
                        add=True)
        return carry2

      lax.fori_loop(0, _CH, dbody, 0)

    plsc.subcore_barrier()

    pltpu.sync_copy(acc_sh.at[pl.ds(s * _RPS, _RPS)],
                    out_hbm.at[c, pl.ds(s * _RPS, _RPS)])

    @pl.when(jnp.logical_and(have_deg, s < _N // _DST))
    def _():
      pltpu.sync_copy(dacc_sh.at[pl.ds(s * _DST, _DST)],
                      deg_hbm.at[c, pl.ds(s * _DST, _DST)])

  return k


def _rows(d):
  return pl.BlockSpec((_BM, d), lambda i: (i, 0))


def _part(d):
  return pl.BlockSpec((_NC, _BM, d), lambda i: (0, i, 0))


def _full(r, c):
  return pl.BlockSpec((r, c), lambda i: (0, 0))


def _out(d):
  return jax.ShapeDtypeStruct((_N, d), jnp.float32)


def _mm_body(x_ref, w_ref, o_ref):
  o_ref[...] = jnp.dot(x_ref[...], w_ref[...],
                       preferred_element_type=jnp.float32)


def _tc_mm(x, w):
  fin, fout = w.shape
  return pl.pallas_call(
      _mm_body,
      grid=(_GRID,),
      in_specs=[_rows(fin), _full(fin, fout)],
      out_specs=_rows(fout),
      out_shape=_out(fout),
  )(x, w)


def _mm_bias_body(*refs):
  npairs = (len(refs) - 2) // 2
  hs, ws, b, o = refs[:npairs], refs[npairs:2 * npairs], refs[-2], refs[-1]
  acc = b[...]
  for h, w in zip(hs, ws):
    acc = acc + jnp.dot(h[...], w[...], preferred_element_type=jnp.float32)
  o[...] = acc


def _tc_mm_bias(hs, ws, b, fout):
  return pl.pallas_call(
      _mm_bias_body,
      grid=(_GRID,),
      in_specs=[_rows(h.shape[1]) for h in hs]
               + [_full(w.shape[0], fout) for w in ws] + [_full(1, fout)],
      out_specs=_rows(fout),
      out_shape=_out(fout),
  )(*hs, *ws, b)


def _c0_body(degp, p, g, wl1, h1_o, y1_o, inv_o):
  deg = degp[0] + degp[1]
  inv = 1.0 / jnp.maximum(deg, 1.0)
  h1 = jnp.maximum((p[0] + p[1]) * inv + g[...], 0.0)
  h1_o[...] = h1
  y1_o[...] = jnp.dot(h1, wl1[...], preferred_element_type=jnp.float32)
  inv_o[...] = inv


def _c1_body(p, inv, g, o):
  o[...] = jnp.maximum((p[0] + p[1]) * inv[...] + g[...], 0.0)


def _c2_body(p, inv, g, wl, oa, ob):
  agg = (p[0] + p[1]) * inv[...]
  h3 = jnp.maximum(
      jnp.dot(agg, wl[...], preferred_element_type=jnp.float32) + g[...], 0.0)
  oa[...] = h3[:, :64]
  ob[...] = h3[:, 64:]


def _c3_body(pa, pb, inv, g, wla, wlb, oa, ob, oc, od):
  iv = inv[...]
  h4 = jnp.maximum(
      jnp.dot((pa[0] + pa[1]) * iv, wla[...], preferred_element_type=jnp.float32)
      + jnp.dot((pb[0] + pb[1]) * iv, wlb[...], preferred_element_type=jnp.float32)
      + g[...], 0.0)
  oa[...] = h4[:, :64]
  ob[...] = h4[:, 64:128]
  oc[...] = h4[:, 128:192]
  od[...] = h4[:, 192:]


def _c4_body(pa, pb, pc, pd, inv, g, wla, wlb, wlc, wld, wout, bout, o):
  iv = inv[...]
  h5 = (jnp.dot((pa[0] + pa[1]) * iv, wla[...], preferred_element_type=jnp.float32)
        + jnp.dot((pb[0] + pb[1]) * iv, wlb[...], preferred_element_type=jnp.float32)
        + jnp.dot((pc[0] + pc[1]) * iv, wlc[...], preferred_element_type=jnp.float32)
        + jnp.dot((pd[0] + pd[1]) * iv, wld[...], preferred_element_type=jnp.float32)
        + g[...])
  h5 = jnp.maximum(h5, 0.0)
  logits = jnp.dot(h5, wout[...], preferred_element_type=jnp.float32) + bout[...]
  m = jnp.max(logits, axis=-1, keepdims=True)
  e = jnp.exp(logits - m)
  o[...] = e / jnp.sum(e, axis=-1, keepdims=True)


def kernel(x, edge_index, Wl0, Wr0, b0, Wl1, Wr1, b1, Wl2, Wr2, b2,
           Wl3, Wr3, b3, Wl4, Wr4, b4, Wout, bout):
  src2 = edge_index[0].reshape(_E // _B, _B)
  dst2 = edge_index[1].reshape(_E // _B, _B)
  b0r, b1r, b2r, b3r, b4r = (v.reshape(1, -1) for v in (b0, b1, b2, b3, b4))
  boutr = bout.reshape(1, -1)

  f1 = jnp.ones((16,), jnp.int32)
  f0 = jnp.zeros((16,), jnp.int32)
  agg = _sc_agg(64)

  y0 = _tc_mm(x, Wl0)
  p0, degp = agg(src2, dst2, y0, f1)
  g0 = _tc_mm_bias([x], [Wr0], b0r, 64)
  degp = degp.reshape(_NC, _N, 1)
  h1, y1, inv = pl.pallas_call(
      _c0_body,
      grid=(_GRID,),
      in_specs=[_part(1), _part(64), _rows(64), _full(64, 64)],
      out_specs=[_rows(64), _rows(64), _rows(1)],
      out_shape=[_out(64), _out(64), _out(1)],
  )(degp, p0, g0, Wl1)

  p1, _ = agg(src2, dst2, y1, f0)
  g1 = _tc_mm_bias([h1], [Wr1], b1r, 64)
  h2 = pl.pallas_call(
      _c1_body,
      grid=(_GRID,),
      in_specs=[_part(64), _rows(1), _rows(64)],
      out_specs=_rows(64),
      out_shape=_out(64),
  )(p1, inv, g1)

  p2, _ = agg(src2, dst2, h2, f0)
  g2 = _tc_mm_bias([h2], [Wr2], b2r, 128)
  h3a, h3b = pl.pallas_call(
      _c2_body,
      grid=(_GRID,),
      in_specs=[_part(64), _rows(1), _rows(128), _full(64, 128)],
      out_specs=[_rows(64), _rows(64)],
      out_shape=[_out(64), _out(64)],
  )(p2, inv, g2, Wl2)

  p3a, _ = agg(src2, dst2, h3a, f0)
  p3b, _ = agg(src2, dst2, h3b, f0)
  g3 = _tc_mm_bias([h3a, h3b], [Wr3[:64], Wr3[64:]], b3r, 256)
  h4 = pl.pallas_call(
      _c3_body,
      grid=(_GRID,),
      in_specs=[_part(64), _part(64), _rows(1), _rows(256),
                _full(64, 256), _full(64, 256)],
      out_specs=[_rows(64)] * 4,
      out_shape=[_out(64)] * 4,
  )(p3a, p3b, inv, g3, Wl3[:64], Wl3[64:])

  p4 = [agg(src2, dst2, hq, f0)[0] for hq in h4]
  g4 = _tc_mm_bias(list(h4), [Wr4[:64], Wr4[64:128], Wr4[128:192],
                              Wr4[192:]], b4r, 512)
  out = pl.pallas_call(
      _c4_body,
      grid=(_GRID,),
      in_specs=[_part(64)] * 4 + [_rows(1), _rows(512)]
               + [_full(64, 512)] * 4 + [_full(512, 4), _full(1, 4)],
      out_specs=_rows(4),
      out_shape=jax.ShapeDtypeStruct((_N, 4), jnp.float32),
  )(p4[0], p4[1], p4[2], p4[3], inv, g4,
    Wl4[:64], Wl4[64:128], Wl4[128:192], Wl4[192:],
    Wout, boutr)
  return out


# speedup vs baseline: 1.3153x; 1.0088x over previous
import functools

import jax
import jax.numpy as jnp
from jax import lax
from jax.experimental import pallas as pl
from jax.experimental.pallas import tpu as pltpu
from jax.experimental.pallas import tpu_sc as plsc

_N = 10000
_E = 320000
_B = 125
_NC, _NS = 2, 16
_TILES = _NC * _NS
_EPT = _E // _TILES
_CH = _EPT // _B
_RPS = _N // _NS
_ZCH = _RPS // _B
_BM = 1000
_GRID = _N // _BM


_DST = 2000
_NB = 4


@functools.lru_cache(maxsize=None)
def _sc_agg(W):
  mesh = plsc.VectorSubcoreMesh(
      core_axis_name="c", subcore_axis_name="s", num_cores=_NC, num_subcores=_NS)

  out_type = (jax.ShapeDtypeStruct((_NC, _N, W), jnp.float32),
              jax.ShapeDtypeStruct((_NC, _N, W), jnp.float32),
              jax.ShapeDtypeStruct((_NC, _N), jnp.float32))
  scratch = [
      pltpu.VMEM((_CH, _B), jnp.int32),
      pltpu.VMEM((_CH, _B), jnp.int32),
      pltpu.VMEM((_NB, _B, W), jnp.float32),
      pltpu.VMEM((_B, W), jnp.float32),
      pltpu.VMEM_SHARED((_N, W), jnp.float32),
  ] + [pltpu.SemaphoreType.DMA] * (2 * _NB) + [
      pltpu.VMEM((16,), jnp.int32),
      pltpu.VMEM((16,), jnp.int32),
      pltpu.VMEM((128,), jnp.float32),
      pltpu.VMEM((_DST,), jnp.float32),
      pltpu.VMEM_SHARED((_N,), jnp.float32),
  ]

  @functools.partial(
      pl.kernel,
      out_type=out_type,
      mesh=mesh,
      scratch_types=scratch,
      compiler_params=pltpu.CompilerParams(
          use_tc_tiling_on_sc=False, needs_layout_passes=False),
  )
  def k(src_hbm, dst_hbm, tablea_hbm, tableb_hbm, tflag_hbm, dflag_hbm,
        *rest):
    (outa_hbm, outb_hbm, deg_hbm, src_v, dst_v, rows_v, stage_v,
     acc_sh) = rest[:8]
    gsem = rest[8:8 + _NB]
    ssem = rest[8 + _NB:8 + 2 * _NB]
    (tflag_v, dflag_v, ones_v, dstage_v, dacc_sh) = rest[8 + 2 * _NB:]
    c = lax.axis_index("c")
    s = lax.axis_index("s")
    wid = s * _NC + c
    pltpu.sync_copy(src_hbm.at[pl.ds(wid * _CH, _CH)], src_v)
    pltpu.sync_copy(dst_hbm.at[pl.ds(wid * _CH, _CH)], dst_v)
    pltpu.sync_copy(tflag_hbm, tflag_v)
    pltpu.sync_copy(dflag_hbm, dflag_v)
    two_tables = jnp.sum(tflag_v[...]) > 0
    have_deg = jnp.sum(dflag_v[...]) > 0

    zero16 = jnp.zeros((16,), jnp.float32)

    def zrow(i, carry):
      for j in range(W // 16):
        stage_v[i, pl.ds(j * 16, 16)] = zero16
      return carry

    lax.fori_loop(0, _B, zrow, 0)

    @pl.when(have_deg)
    def _():
      one16 = jnp.ones((16,), jnp.float32)

      def fill1(i, carry):
        ones_v[pl.ds(i * 16, 16)] = one16
        return carry

      lax.fori_loop(0, 128 // 16, fill1, 0)

      def fill0(i, carry):
        dstage_v[pl.ds(i * 16, 16)] = zero16
        return carry

      lax.fori_loop(0, _DST // 16, fill0, 0)

      @pl.when(s < _N // _DST)
      def _():
        pltpu.sync_copy(dstage_v, dacc_sh.at[pl.ds(s * _DST, _DST)])

    def make_phase(table_hbm, out_hbm):
      def issue_gather(g, j):
        pltpu.async_copy(table_hbm.at[src_v.at[g]], rows_v.at[j], gsem[j])

      def wait_gather(g, j):
        pltpu.make_async_copy(
            table_hbm.at[src_v.at[g]], rows_v.at[j], gsem[j]).wait()

      def start_scatter(g, j):
        pltpu.async_copy(rows_v.at[j], acc_sh.at[dst_v.at[g]], ssem[j],
                         add=True)

      def wait_scatter(g, j):
        pltpu.make_async_copy(rows_v.at[j], acc_sh.at[dst_v.at[g]],
                              ssem[j]).wait()

      def phase(with_deg_work):
        for j in range(_NB):
          issue_gather(j, j)

        def zcp(i, carry2):
          pltpu.sync_copy(stage_v, acc_sh.at[pl.ds(s * _RPS + i * _B, _B)])
          return carry2

        lax.fori_loop(0, _ZCH, zcp, 0)
        plsc.subcore_barrier()

        def body(i, carry2):
          g = _NB * i
          for j in range(_NB):
            wait_gather(g + j, j)
            start_scatter(g + j, j)
          for j in range(_NB):
            wait_scatter(g + j, j)
            issue_gather(g + _NB + j, j)
          return carry2

        lax.fori_loop(0, _CH // _NB - 1, body, 0)
        gl = _CH - _NB
        for j in range(_NB):
          wait_gather(gl + j, j)
          start_scatter(gl + j, j)
        for j in range(_NB):
          wait_scatter(gl + j, j)

        if with_deg_work:
          @pl.when(have_deg)
          def _():
            def dbody(g, carry2):
              pltpu.sync_copy(ones_v.at[pl.ds(0, _B)],
                              dacc_sh.at[dst_v.at[g]], add=True)
              return carry2

            lax.fori_loop(0, _CH, dbody, 0)

        plsc.subcore_barrier()

        pltpu.sync_copy(acc_sh.at[pl.ds(s * _RPS, _RPS)],
                        out_hbm.at[c, pl.ds(s * _RPS, _RPS)])

      return phase

    make_phase(tablea_hbm, outa_hbm)(True)

    @pl.when(two_tables)
    def _():
      make_phase(tableb_hbm, outb_hbm)(False)

    @pl.when(jnp.logical_and(have_deg, s < _N // _DST))
    def _():
      pltpu.sync_copy(dacc_sh.at[pl.ds(s * _DST, _DST)],
                      deg_hbm.at[c, pl.ds(s * _DST, _DST)])

  return k


def _rows(d):
  return pl.BlockSpec((_BM, d), lambda i: (i, 0))


def _part(d):
  return pl.BlockSpec((_NC, _BM, d), lambda i: (0, i, 0))


def _full(r, c):
  return pl.BlockSpec((r, c), lambda i: (0, 0))


def _out(d):
  return jax.ShapeDtypeStruct((_N, d), jnp.float32)


def _mm_body(x_ref, w_ref, o_ref):
  o_ref[...] = jnp.dot(x_ref[...], w_ref[...],
                       preferred_element_type=jnp.float32)


def _tc_mm(x, w):
  fin, fout = w.shape
  return pl.pallas_call(
      _mm_body,
      grid=(_GRID,),
      in_specs=[_rows(fin), _full(fin, fout)],
      out_specs=_rows(fout),
      out_shape=_out(fout),
  )(x, w)


def _mm_bias_body(*refs):
  npairs = (len(refs) - 2) // 2
  hs, ws, b, o = refs[:npairs], refs[npairs:2 * npairs], refs[-2], refs[-1]
  acc = b[...]
  for h, w in zip(hs, ws):
    acc = acc + jnp.dot(h[...], w[...], preferred_element_type=jnp.float32)
  o[...] = acc


def _tc_mm_bias(hs, ws, b, fout):
  return pl.pallas_call(
      _mm_bias_body,
      grid=(_GRID,),
      in_specs=[_rows(h.shape[1]) for h in hs]
               + [_full(w.shape[0], fout) for w in ws] + [_full(1, fout)],
      out_specs=_rows(fout),
      out_shape=_out(fout),
  )(*hs, *ws, b)


def _c0_body(degp, p, g, wl1, h1_o, y1_o, inv_o):
  deg = degp[0] + degp[1]
  inv = 1.0 / jnp.maximum(deg, 1.0)
  h1 = jnp.maximum((p[0] + p[1]) * inv + g[...], 0.0)
  h1_o[...] = h1
  y1_o[...] = jnp.dot(h1, wl1[...], preferred_element_type=jnp.float32)
  inv_o[...] = inv


def _c1_body(p, inv, g, o):
  o[...] = jnp.maximum((p[0] + p[1]) * inv[...] + g[...], 0.0)


def _c2_body(p, inv, g, wl, oa, ob):
  agg = (p[0] + p[1]) * inv[...]
  h3 = jnp.maximum(
      jnp.dot(agg, wl[...], preferred_element_type=jnp.float32) + g[...], 0.0)
  oa[...] = h3[:, :64]
  ob[...] = h3[:, 64:]


def _c3_body(pa, pb, inv, g, wla, wlb, oa, ob, oc, od):
  iv = inv[...]
  h4 = jnp.maximum(
      jnp.dot((pa[0] + pa[1]) * iv, wla[...], preferred_element_type=jnp.float32)
      + jnp.dot((pb[0] + pb[1]) * iv, wlb[...], preferred_element_type=jnp.float32)
      + g[...], 0.0)
  oa[...] = h4[:, :64]
  ob[...] = h4[:, 64:128]
  oc[...] = h4[:, 128:192]
  od[...] = h4[:, 192:]


def _c4_body(pa, pb, pc, pd, inv, g, wla, wlb, wlc, wld, wout, bout, o):
  iv = inv[...]
  h5 = (jnp.dot((pa[0] + pa[1]) * iv, wla[...], preferred_element_type=jnp.float32)
        + jnp.dot((pb[0] + pb[1]) * iv, wlb[...], preferred_element_type=jnp.float32)
        + jnp.dot((pc[0] + pc[1]) * iv, wlc[...], preferred_element_type=jnp.float32)
        + jnp.dot((pd[0] + pd[1]) * iv, wld[...], preferred_element_type=jnp.float32)
        + g[...])
  h5 = jnp.maximum(h5, 0.0)
  logits = jnp.dot(h5, wout[...], preferred_element_type=jnp.float32) + bout[...]
  m = jnp.max(logits, axis=-1, keepdims=True)
  e = jnp.exp(logits - m)
  o[...] = e / jnp.sum(e, axis=-1, keepdims=True)


def kernel(x, edge_index, Wl0, Wr0, b0, Wl1, Wr1, b1, Wl2, Wr2, b2,
           Wl3, Wr3, b3, Wl4, Wr4, b4, Wout, bout):
  src2 = edge_index[0].reshape(_E // _B, _B)
  dst2 = edge_index[1].reshape(_E // _B, _B)
  b0r, b1r, b2r, b3r, b4r = (v.reshape(1, -1) for v in (b0, b1, b2, b3, b4))
  boutr = bout.reshape(1, -1)

  f1 = jnp.ones((16,), jnp.int32)
  f0 = jnp.zeros((16,), jnp.int32)
  agg2 = _sc_agg(64)

  def agg(ta, tb, two, dflag):
    return agg2(src2, dst2, ta, tb, two, dflag)

  y0 = _tc_mm(x, Wl0)
  p0, _, degp = agg(y0, y0, f0, f1)
  g0 = _tc_mm_bias([x], [Wr0], b0r, 64)
  degp = degp.reshape(_NC, _N, 1)
  h1, y1, inv = pl.pallas_call(
      _c0_body,
      grid=(_GRID,),
      in_specs=[_part(1), _part(64), _rows(64), _full(64, 64)],
      out_specs=[_rows(64), _rows(64), _rows(1)],
      out_shape=[_out(64), _out(64), _out(1)],
  )(degp, p0, g0, Wl1)

  p1, _, _ = agg(y1, y1, f0, f0)
  g1 = _tc_mm_bias([h1], [Wr1], b1r, 64)
  h2 = pl.pallas_call(
      _c1_body,
      grid=(_GRID,),
      in_specs=[_part(64), _rows(1), _rows(64)],
      out_specs=_rows(64),
      out_shape=_out(64),
  )(p1, inv, g1)

  p2, _, _ = agg(h2, h2, f0, f0)
  g2 = _tc_mm_bias([h2], [Wr2], b2r, 128)
  h3a, h3b = pl.pallas_call(
      _c2_body,
      grid=(_GRID,),
      in_specs=[_part(64), _rows(1), _rows(128), _full(64, 128)],
      out_specs=[_rows(64), _rows(64)],
      out_shape=[_out(64), _out(64)],
  )(p2, inv, g2, Wl2)

  p3a, p3b, _ = agg(h3a, h3b, f1, f0)
  g3 = _tc_mm_bias([h3a, h3b], [Wr3[:64], Wr3[64:]], b3r, 256)
  h4 = pl.pallas_call(
      _c3_body,
      grid=(_GRID,),
      in_specs=[_part(64), _part(64), _rows(1), _rows(256),
                _full(64, 256), _full(64, 256)],
      out_specs=[_rows(64)] * 4,
      out_shape=[_out(64)] * 4,
  )(p3a, p3b, inv, g3, Wl3[:64], Wl3[64:])

  p4ab = agg(h4[0], h4[1], f1, f0)
  p4cd = agg(h4[2], h4[3], f1, f0)
  p4 = [p4ab[0], p4ab[1], p4cd[0], p4cd[1]]
  g4 = _tc_mm_bias(list(h4), [Wr4[:64], Wr4[64:128], Wr4[128:192],
                              Wr4[192:]], b4r, 512)
  out = pl.pallas_call(
      _c4_body,
      grid=(_GRID,),
      in_specs=[_part(64)] * 4 + [_rows(1), _rows(512)]
               + [_full(64, 512)] * 4 + [_full(512, 4), _full(1, 4)],
      out_specs=_rows(4),
      out_shape=jax.ShapeDtypeStruct((_N, 4), jnp.float32),
  )(p4[0], p4[1], p4[2], p4[3], inv, g4,
    Wl4[:64], Wl4[64:128], Wl4[128:192], Wl4[192:],
    Wout, boutr)
  return out


# speedup vs baseline: 1.3294x; 1.0108x over previous
import functools

import jax
import jax.numpy as jnp
from jax import lax
from jax.experimental import pallas as pl
from jax.experimental.pallas import tpu as pltpu
from jax.experimental.pallas import tpu_sc as plsc

_N = 10000
_E = 320000
_B = 125
_NC, _NS = 2, 16
_TILES = _NC * _NS
_EPT = _E // _TILES
_CH = _EPT // _B
_RPS = _N // _NS
_ZCH = _RPS // _B
_BM = 2000
_GRID = _N // _BM


_DST = 2000
_NB = 4


@functools.lru_cache(maxsize=None)
def _sc_agg(W):
  mesh = plsc.VectorSubcoreMesh(
      core_axis_name="c", subcore_axis_name="s", num_cores=_NC, num_subcores=_NS)

  out_type = (jax.ShapeDtypeStruct((_NC, _N, W), jnp.float32),
              jax.ShapeDtypeStruct((_NC, _N, W), jnp.float32),
              jax.ShapeDtypeStruct((_NC, _N), jnp.float32))
  scratch = [
      pltpu.VMEM((_CH, _B), jnp.int32),
      pltpu.VMEM((_CH, _B), jnp.int32),
      pltpu.VMEM((_NB, _B, W), jnp.float32),
      pltpu.VMEM((_B, W), jnp.float32),
      pltpu.VMEM_SHARED((_N, W), jnp.float32),
  ] + [pltpu.SemaphoreType.DMA] * (2 * _NB) + [
      pltpu.VMEM((16,), jnp.int32),
      pltpu.VMEM((16,), jnp.int32),
      pltpu.VMEM((128,), jnp.float32),
      pltpu.VMEM((_DST,), jnp.float32),
      pltpu.VMEM_SHARED((_N,), jnp.float32),
  ]

  @functools.partial(
      pl.kernel,
      out_type=out_type,
      mesh=mesh,
      scratch_types=scratch,
      compiler_params=pltpu.CompilerParams(
          use_tc_tiling_on_sc=False, needs_layout_passes=False),
  )
  def k(src_hbm, dst_hbm, tablea_hbm, tableb_hbm, tflag_hbm, dflag_hbm,
        *rest):
    (outa_hbm, outb_hbm, deg_hbm, src_v, dst_v, rows_v, stage_v,
     acc_sh) = rest[:8]
    gsem = rest[8:8 + _NB]
    ssem = rest[8 + _NB:8 + 2 * _NB]
    (tflag_v, dflag_v, ones_v, dstage_v, dacc_sh) = rest[8 + 2 * _NB:]
    c = lax.axis_index("c")
    s = lax.axis_index("s")
    wid = s * _NC + c
    pltpu.sync_copy(src_hbm.at[pl.ds(wid * _CH, _CH)], src_v)
    pltpu.sync_copy(dst_hbm.at[pl.ds(wid * _CH, _CH)], dst_v)
    pltpu.sync_copy(tflag_hbm, tflag_v)
    pltpu.sync_copy(dflag_hbm, dflag_v)
    two_tables = jnp.sum(tflag_v[...]) > 0
    have_deg = jnp.sum(dflag_v[...]) > 0

    zero16 = jnp.zeros((16,), jnp.float32)

    def zrow(i, carry):
      for j in range(W // 16):
        stage_v[i, pl.ds(j * 16, 16)] = zero16
      return carry

    lax.fori_loop(0, _B, zrow, 0)

    @pl.when(have_deg)
    def _():
      one16 = jnp.ones((16,), jnp.float32)

      def fill1(i, carry):
        ones_v[pl.ds(i * 16, 16)] = one16
        return carry

      lax.fori_loop(0, 128 // 16, fill1, 0)

      def fill0(i, carry):
        dstage_v[pl.ds(i * 16, 16)] = zero16
        return carry

      lax.fori_loop(0, _DST // 16, fill0, 0)

      @pl.when(s < _N // _DST)
      def _():
        pltpu.sync_copy(dstage_v, dacc_sh.at[pl.ds(s * _DST, _DST)])

    def make_phase(table_hbm, out_hbm):
      def issue_gather(g, j):
        pltpu.async_copy(table_hbm.at[src_v.at[g]], rows_v.at[j], gsem[j])

      def wait_gather(g, j):
        pltpu.make_async_copy(
            table_hbm.at[src_v.at[g]], rows_v.at[j], gsem[j]).wait()

      def start_scatter(g, j):
        pltpu.async_copy(rows_v.at[j], acc_sh.at[dst_v.at[g]], ssem[j],
                         add=True)

      def wait_scatter(g, j):
        pltpu.make_async_copy(rows_v.at[j], acc_sh.at[dst_v.at[g]],
                              ssem[j]).wait()

      def phase(with_deg_work):
        for j in range(_NB):
          issue_gather(j, j)

        def zcp(i, carry2):
          pltpu.sync_copy(stage_v, acc_sh.at[pl.ds(s * _RPS + i * _B, _B)])
          return carry2

        lax.fori_loop(0, _ZCH, zcp, 0)
        plsc.subcore_barrier()

        def body(i, carry2):
          g = _NB * i
          for j in range(_NB):
            wait_gather(g + j, j)
            start_scatter(g + j, j)
          for j in range(_NB):
            wait_scatter(g + j, j)
            issue_gather(g + _NB + j, j)
          return carry2

        lax.fori_loop(0, _CH // _NB - 1, body, 0)
        gl = _CH - _NB
        for j in range(_NB):
          wait_gather(gl + j, j)
          start_scatter(gl + j, j)
        for j in range(_NB):
          wait_scatter(gl + j, j)

        if with_deg_work:
          @pl.when(have_deg)
          def _():
            def dbody(g, carry2):
              pltpu.sync_copy(ones_v.at[pl.ds(0, _B)],
                              dacc_sh.at[dst_v.at[g]], add=True)
              return carry2

            lax.fori_loop(0, _CH, dbody, 0)

        plsc.subcore_barrier()

        pltpu.sync_copy(acc_sh.at[pl.ds(s * _RPS, _RPS)],
                        out_hbm.at[c, pl.ds(s * _RPS, _RPS)])

      return phase

    make_phase(tablea_hbm, outa_hbm)(True)

    @pl.when(two_tables)
    def _():
      make_phase(tableb_hbm, outb_hbm)(False)

    @pl.when(jnp.logical_and(have_deg, s < _N // _DST))
    def _():
      pltpu.sync_copy(dacc_sh.at[pl.ds(s * _DST, _DST)],
                      deg_hbm.at[c, pl.ds(s * _DST, _DST)])

  return k


def _rows(d):
  return pl.BlockSpec((_BM, d), lambda i: (i, 0))


def _part(d):
  return pl.BlockSpec((_NC, _BM, d), lambda i: (0, i, 0))


def _full(r, c):
  return pl.BlockSpec((r, c), lambda i: (0, 0))


def _out(d):
  return jax.ShapeDtypeStruct((_N, d), jnp.float32)


def _mm_body(x_ref, w_ref, o_ref):
  o_ref[...] = jnp.dot(x_ref[...], w_ref[...],
                       preferred_element_type=jnp.float32)


def _tc_mm(x, w):
  fin, fout = w.shape
  return pl.pallas_call(
      _mm_body,
      grid=(_GRID,),
      in_specs=[_rows(fin), _full(fin, fout)],
      out_specs=_rows(fout),
      out_shape=_out(fout),
  )(x, w)


def _mm_bias_body(*refs):
  npairs = (len(refs) - 2) // 2
  hs, ws, b, o = refs[:npairs], refs[npairs:2 * npairs], refs[-2], refs[-1]
  acc = b[...]
  for h, w in zip(hs, ws):
    acc = acc + jnp.dot(h[...], w[...], preferred_element_type=jnp.float32)
  o[...] = acc


def _tc_mm_bias(hs, ws, b, fout):
  return pl.pallas_call(
      _mm_bias_body,
      grid=(_GRID,),
      in_specs=[_rows(h.shape[1]) for h in hs]
               + [_full(w.shape[0], fout) for w in ws] + [_full(1, fout)],
      out_specs=_rows(fout),
      out_shape=_out(fout),
  )(*hs, *ws, b)


def _c0_body(degp, p, g, wl1, h1_o, y1_o, inv_o):
  deg = degp[0] + degp[1]
  inv = 1.0 / jnp.maximum(deg, 1.0)
  h1 = jnp.maximum((p[0] + p[1]) * inv + g[...], 0.0)
  h1_o[...] = h1
  y1_o[...] = jnp.dot(h1, wl1[...], preferred_element_type=jnp.float32)
  inv_o[...] = inv


def _c1_body(p, inv, g, o):
  o[...] = jnp.maximum((p[0] + p[1]) * inv[...] + g[...], 0.0)


def _c2_body(p, inv, g, wl, oa, ob):
  agg = (p[0] + p[1]) * inv[...]
  h3 = jnp.maximum(
      jnp.dot(agg, wl[...], preferred_element_type=jnp.float32) + g[...], 0.0)
  oa[...] = h3[:, :64]
  ob[...] = h3[:, 64:]


def _c3_body(pa, pb, inv, g, wla, wlb, oa, ob, oc, od):
  iv = inv[...]
  h4 = jnp.maximum(
      jnp.dot((pa[0] + pa[1]) * iv, wla[...], preferred_element_type=jnp.float32)
      + jnp.dot((pb[0] + pb[1]) * iv, wlb[...], preferred_element_type=jnp.float32)
      + g[...], 0.0)
  oa[...] = h4[:, :64]
  ob[...] = h4[:, 64:128]
  oc[...] = h4[:, 128:192]
  od[...] = h4[:, 192:]


def _c4_body(pa, pb, pc, pd, inv, g, wla, wlb, wlc, wld, wout, bout, o):
  iv = inv[...]
  h5 = (jnp.dot((pa[0] + pa[1]) * iv, wla[...], preferred_element_type=jnp.float32)
        + jnp.dot((pb[0] + pb[1]) * iv, wlb[...], preferred_element_type=jnp.float32)
        + jnp.dot((pc[0] + pc[1]) * iv, wlc[...], preferred_element_type=jnp.float32)
        + jnp.dot((pd[0] + pd[1]) * iv, wld[...], preferred_element_type=jnp.float32)
        + g[...])
  h5 = jnp.maximum(h5, 0.0)
  logits = jnp.dot(h5, wout[...], preferred_element_type=jnp.float32) + bout[...]
  m = jnp.max(logits, axis=-1, keepdims=True)
  e = jnp.exp(logits - m)
  o[...] = e / jnp.sum(e, axis=-1, keepdims=True)


def kernel(x, edge_index, Wl0, Wr0, b0, Wl1, Wr1, b1, Wl2, Wr2, b2,
           Wl3, Wr3, b3, Wl4, Wr4, b4, Wout, bout):
  src2 = edge_index[0].reshape(_E // _B, _B)
  dst2 = edge_index[1].reshape(_E // _B, _B)
  b0r, b1r, b2r, b3r, b4r = (v.reshape(1, -1) for v in (b0, b1, b2, b3, b4))
  boutr = bout.reshape(1, -1)

  f1 = jnp.ones((16,), jnp.int32)
  f0 = jnp.zeros((16,), jnp.int32)
  agg2 = _sc_agg(64)

  def agg(ta, tb, two, dflag):
    return agg2(src2, dst2, ta, tb, two, dflag)

  y0 = _tc_mm(x, Wl0)
  p0, _, degp = agg(y0, y0, f0, f1)
  g0 = _tc_mm_bias([x], [Wr0], b0r, 64)
  degp = degp.reshape(_NC, _N, 1)
  h1, y1, inv = pl.pallas_call(
      _c0_body,
      grid=(_GRID,),
      in_specs=[_part(1), _part(64), _rows(64), _full(64, 64)],
      out_specs=[_rows(64), _rows(64), _rows(1)],
      out_shape=[_out(64), _out(64), _out(1)],
  )(degp, p0, g0, Wl1)

  p1, _, _ = agg(y1, y1, f0, f0)
  g1 = _tc_mm_bias([h1], [Wr1], b1r, 64)
  h2 = pl.pallas_call(
      _c1_body,
      grid=(_GRID,),
      in_specs=[_part(64), _rows(1), _rows(64)],
      out_specs=_rows(64),
      out_shape=_out(64),
  )(p1, inv, g1)

  p2, _, _ = agg(h2, h2, f0, f0)
  g2 = _tc_mm_bias([h2], [Wr2], b2r, 128)
  h3a, h3b = pl.pallas_call(
      _c2_body,
      grid=(_GRID,),
      in_specs=[_part(64), _rows(1), _rows(128), _full(64, 128)],
      out_specs=[_rows(64), _rows(64)],
      out_shape=[_out(64), _out(64)],
  )(p2, inv, g2, Wl2)

  p3a, p3b, _ = agg(h3a, h3b, f1, f0)
  g3 = _tc_mm_bias([h3a, h3b], [Wr3[:64], Wr3[64:]], b3r, 256)
  h4 = pl.pallas_call(
      _c3_body,
      grid=(_GRID,),
      in_specs=[_part(64), _part(64), _rows(1), _rows(256),
                _full(64, 256), _full(64, 256)],
      out_specs=[_rows(64)] * 4,
      out_shape=[_out(64)] * 4,
  )(p3a, p3b, inv, g3, Wl3[:64], Wl3[64:])

  p4ab = agg(h4[0], h4[1], f1, f0)
  p4cd = agg(h4[2], h4[3], f1, f0)
  p4 = [p4ab[0], p4ab[1], p4cd[0], p4cd[1]]
  g4 = _tc_mm_bias(list(h4), [Wr4[:64], Wr4[64:128], Wr4[128:192],
                              Wr4[192:]], b4r, 512)
  out = pl.pallas_call(
      _c4_body,
      grid=(_GRID,),
      in_specs=[_part(64)] * 4 + [_rows(1), _rows(512)]
               + [_full(64, 512)] * 4 + [_full(512, 4), _full(1, 4)],
      out_specs=_rows(4),
      out_shape=jax.ShapeDtypeStruct((_N, 4), jnp.float32),
  )(p4[0], p4[1], p4[2], p4[3], inv, g4,
    Wl4[:64], Wl4[64:128], Wl4[128:192], Wl4[192:],
    Wout, boutr)
  return out
